# Initial kernel scaffold; baseline (speedup 1.0000x reference)
import jax
import jax.numpy as jnp
from jax.experimental import pallas as pl


def kernel(x):
    raise NotImplementedError("write your pallas kernel here")


# speedup vs baseline: 31.1763x; 31.1763x over previous
import jax
import jax.numpy as jnp
from jax.experimental import pallas as pl
from jax.experimental.pallas import tpu as pltpu

_MASK = 0x7FFFFFFF
_INT_MIN = -2147483648
_INT_MAX = 2147483647


def _to_key(b):
    return jnp.where(b < 0, b ^ jnp.int32(_MASK), b)


def _median_body(k1, k2, x_ref, o_ref, key_ref):
    bits = jax.lax.bitcast_convert_type(x_ref[...], jnp.int32)
    key = _to_key(bits)
    key_ref[...] = key

    c0 = jnp.sum((key < 0).astype(jnp.int32), axis=1, keepdims=True)
    p = jnp.where(c0 <= k1, jnp.zeros_like(c0),
                  jnp.full_like(c0, jnp.int32(_INT_MIN)))

    def body(i, p):
        t = p + (jnp.int32(1) << (jnp.int32(30) - i))
        c = jnp.sum((key_ref[...] < t).astype(jnp.int32), axis=1,
                    keepdims=True)
        return jnp.where(c <= k1, t, p)

    p = jax.lax.fori_loop(0, 31, body, p)

    keys = key_ref[...]
    le = jnp.sum((keys <= p).astype(jnp.int32), axis=1, keepdims=True)
    gt_min = jnp.min(jnp.where(keys > p, keys, jnp.int32(_INT_MAX)), axis=1,
                     keepdims=True)
    p2 = jnp.where(le >= k2 + 1, p, gt_min)

    f1 = jax.lax.bitcast_convert_type(_to_key(p), jnp.float32)
    f2 = jax.lax.bitcast_convert_type(_to_key(p2), jnp.float32)
    o_ref[...] = f1 + (f2 - f1) * jnp.float32(0.5)


def kernel(x):
    N, C, H, W = x.shape
    rows = N * C
    hw = H * W
    k1 = (hw - 1) // 2
    k2 = hw // 2

    if rows % 16 == 0:
        r = 16
    elif rows % 8 == 0:
        r = 8
    else:
        r = rows

    x2 = x.reshape(rows, hw)
    out = pl.pallas_call(
        lambda x_ref, o_ref, key_ref: _median_body(k1, k2, x_ref, o_ref,
                                                   key_ref),
        grid=(rows // r,),
        in_specs=[pl.BlockSpec((r, hw), lambda i: (i, 0))],
        out_specs=pl.BlockSpec((r, 1), lambda i: (i, 0)),
        out_shape=jax.ShapeDtypeStruct((rows, 1), jnp.float32),
        scratch_shapes=[pltpu.VMEM((r, hw), jnp.int32)],
        compiler_params=pltpu.CompilerParams(
            dimension_semantics=("parallel",)),
    )(x2)
    return out.reshape(N, C, 1, 1)


# speedup vs baseline: 35.5861x; 1.1414x over previous
import jax
import jax.numpy as jnp
from jax.experimental import pallas as pl
from jax.experimental.pallas import tpu as pltpu

_MASK = 0x7FFFFFFF
_INT_MIN = -2147483648
_INT_MAX = 2147483647


def _to_key(b):
    return jnp.where(b < 0, b ^ jnp.int32(_MASK), b)


def _median_body(k1, k2, x_ref, o_ref, key_ref):
    bits = jax.lax.bitcast_convert_type(x_ref[...], jnp.int32)
    key = _to_key(bits)
    key_ref[...] = key

    c0 = jnp.sum((key < 0).astype(jnp.int32), axis=1, keepdims=True)
    p = jnp.where(c0 <= k1, jnp.zeros_like(c0),
                  jnp.full_like(c0, jnp.int32(_INT_MIN)))

    def body(i, p):
        t = p + (jnp.int32(1) << (jnp.int32(30) - i))
        c = jnp.sum((key_ref[...] < t).astype(jnp.int32), axis=1,
                    keepdims=True)
        return jnp.where(c <= k1, t, p)

    p = jax.lax.fori_loop(0, 31, body, p)

    keys = key_ref[...]
    le = jnp.sum((keys <= p).astype(jnp.int32), axis=1, keepdims=True)
    gt_min = jnp.min(jnp.where(keys > p, keys, jnp.int32(_INT_MAX)), axis=1,
                     keepdims=True)
    p2 = jnp.where(le >= k2 + 1, p, gt_min)

    f1 = jax.lax.bitcast_convert_type(_to_key(p), jnp.float32)
    f2 = jax.lax.bitcast_convert_type(_to_key(p2), jnp.float32)
    o_ref[...] = f1 + (f2 - f1) * jnp.float32(0.5)


def kernel(x):
    N, C, H, W = x.shape
    rows = N * C
    hw = H * W
    k1 = (hw - 1) // 2
    k2 = hw // 2

    if rows % 32 == 0:
        r = 32
    elif rows % 16 == 0:
        r = 16
    elif rows % 8 == 0:
        r = 8
    else:
        r = rows

    x2 = x.reshape(rows, hw)
    out = pl.pallas_call(
        lambda x_ref, o_ref, key_ref: _median_body(k1, k2, x_ref, o_ref,
                                                   key_ref),
        grid=(rows // r,),
        in_specs=[pl.BlockSpec((r, hw), lambda i: (i, 0))],
        out_specs=pl.BlockSpec((r, 1), lambda i: (i, 0)),
        out_shape=jax.ShapeDtypeStruct((rows, 1), jnp.float32),
        scratch_shapes=[pltpu.VMEM((r, hw), jnp.int32)],
        compiler_params=pltpu.CompilerParams(
            dimension_semantics=("parallel",)),
    )(x2)
    return out.reshape(N, C, 1, 1)
